# baseline (device time: 329149 ns/iter reference)
import jax
import jax.numpy as jnp
from jax import lax
from jax.experimental import pallas as pl
from jax.experimental.pallas import tpu as pltpu

NZ = 4


def kernel(x):
    x = x.astype(jnp.bfloat16)
    m, n = x.shape
    hm = m // 2
    mc = hm // NZ

    def body(x_ref, out_ref, rs_recv, rs_send, send_sems, recv_sems):
        my_x = lax.axis_index("x")
        my_y = lax.axis_index("y")
        my_z = lax.axis_index("z")
        right = (my_z + 1) % NZ
        left = (my_z + NZ - 1) % NZ

        barrier_sem = pltpu.get_barrier_semaphore()
        for nbr in (left, right):
            pl.semaphore_signal(
                barrier_sem, inc=1,
                device_id=(my_x, my_y, nbr),
                device_id_type=pl.DeviceIdType.MESH,
            )
        pl.semaphore_wait(barrier_sem, 2)

        def rows(c, d):
            return pl.ds(d * hm + c * mc, mc)

        for h in range(NZ - 1):
            rdmas = []
            for d in range(2):
                s_idx = (my_z + NZ - h) % NZ if d == 0 else (my_z + h) % NZ
                if h == 0:
                    rs_send[d, 0] = x_ref[rows(s_idx, d), :]
                else:
                    rs_send[d, h] = rs_recv[d, h - 1] + x_ref[rows(s_idx, d), :]
                rdmas.append(pltpu.make_async_remote_copy(
                    src_ref=rs_send.at[d, h],
                    dst_ref=rs_recv.at[d, h],
                    send_sem=send_sems.at[d, h],
                    recv_sem=recv_sems.at[d, h],
                    device_id=(my_x, my_y, right if d == 0 else left),
                    device_id_type=pl.DeviceIdType.MESH,
                ))
            for r in rdmas:
                r.start()
            for r in rdmas:
                r.wait()

        for d in range(2):
            q = (my_z + 1) % NZ if d == 0 else (my_z + NZ - 1) % NZ
            out_ref[rows(q, d), :] = rs_recv[d, NZ - 2] + x_ref[rows(q, d), :]

        for h in range(NZ - 1):
            rdmas = []
            for d in range(2):
                g = (my_z + 1 + NZ - h) % NZ if d == 0 else (my_z + NZ - 1 + h) % NZ
                rdmas.append(pltpu.make_async_remote_copy(
                    src_ref=out_ref.at[rows(g, d), :],
                    dst_ref=out_ref.at[rows(g, d), :],
                    send_sem=send_sems.at[d, NZ - 1 + h],
                    recv_sem=recv_sems.at[d, NZ - 1 + h],
                    device_id=(my_x, my_y, right if d == 0 else left),
                    device_id_type=pl.DeviceIdType.MESH,
                ))
            for r in rdmas:
                r.start()
            for r in rdmas:
                r.wait()

    return pl.pallas_call(
        body,
        out_shape=jax.ShapeDtypeStruct((m, n), jnp.bfloat16),
        in_specs=[pl.BlockSpec(memory_space=pltpu.VMEM)],
        out_specs=pl.BlockSpec(memory_space=pltpu.VMEM),
        scratch_shapes=[
            pltpu.VMEM((2, NZ - 1, mc, n), jnp.bfloat16),
            pltpu.VMEM((2, NZ - 1, mc, n), jnp.bfloat16),
            pltpu.SemaphoreType.DMA((2, 2 * (NZ - 1))),
            pltpu.SemaphoreType.DMA((2, 2 * (NZ - 1))),
        ],
        compiler_params=pltpu.CompilerParams(
            collective_id=0,
            vmem_limit_bytes=100 * 1024 * 1024,
        ),
    )(x)


# device time: 251578 ns/iter; 1.3083x vs baseline; 1.3083x over previous
import jax
import jax.numpy as jnp
from jax import lax
from jax.experimental import pallas as pl
from jax.experimental.pallas import tpu as pltpu

NZ = 4
NP = 4
K = 8
AGK = 2


def kernel(x):
    x = x.astype(jnp.bfloat16)
    m, n = x.shape
    qm = m // NP
    sm = qm // K
    agm = qm // 2 // AGK

    def body(x_ref, out_ref, pf, sf,
             pf_ssem, pf_rsem, sf_ssem, sf_rsem, ag_ssem, ag_rsem):
        my_x = lax.axis_index("x")
        my_y = lax.axis_index("y")
        my_z = lax.axis_index("z")
        q = 2 * my_x + my_y
        qbase = q * qm

        p = jnp.where(my_x == 0, my_y, 3 - my_y)

        def ring_coords(t):
            return (jnp.where(t >= 2, 1, 0),
                    jnp.where((t == 1) | (t == 2), 1, 0))

        rx, ry = ring_coords((p + 1) % NP)
        lx, ly = ring_coords((p + 3) % NP)
        zr = (my_z + 1) % NZ
        zl = (my_z + NZ - 1) % NZ

        barrier_sem = pltpu.get_barrier_semaphore()
        for dev in ((my_x, my_y, zr), (my_x, my_y, zl),
                    (rx, ry, my_z), (lx, ly, my_z)):
            pl.semaphore_signal(
                barrier_sem, inc=1,
                device_id=dev, device_id_type=pl.DeviceIdType.MESH,
            )
        pl.semaphore_wait(barrier_sem, 4)

        def rows(j):
            return pl.ds(qbase + j * sm, sm)

        is_head = my_z == 0
        is_tail = my_z == NZ - 1
        interior = jnp.logical_and(my_z > 0, my_z < NZ - 1)

        pf_d = [pltpu.make_async_remote_copy(
            src_ref=pf.at[j], dst_ref=pf.at[j],
            send_sem=pf_ssem.at[j], recv_sem=pf_rsem.at[j],
            device_id=(my_x, my_y, zr),
            device_id_type=pl.DeviceIdType.MESH) for j in range(K)]
        sf_d = [pltpu.make_async_remote_copy(
            src_ref=sf.at[j], dst_ref=sf.at[j],
            send_sem=sf_ssem.at[j], recv_sem=sf_rsem.at[j],
            device_id=(my_x, my_y, zl),
            device_id_type=pl.DeviceIdType.MESH) for j in range(K)]

        @pl.when(is_head)
        def _():
            for j in range(K):
                pf[j] = x_ref[rows(j), :]
                pf_d[j].start()

        @pl.when(is_tail)
        def _():
            for j in range(K):
                sf[j] = x_ref[rows(j), :]
                sf_d[j].start()

        for j in range(K):
            @pl.when(my_z > 0)
            def _(j=j):
                pf_d[j].wait_recv()
                pf[j] = pf[j] + x_ref[rows(j), :]

            @pl.when(interior)
            def _(j=j):
                pf_d[j].start()

            @pl.when(my_z < NZ - 1)
            def _(j=j):
                sf_d[j].wait_recv()
                out_ref[rows(j), :] = pf[j] + sf[j]

            @pl.when(is_tail)
            def _(j=j):
                out_ref[rows(j), :] = pf[j]

            @pl.when(interior)
            def _(j=j):
                sf[j] = sf[j] + x_ref[rows(j), :]
                sf_d[j].start()

        def ag_desc(d, h, s):
            t = (p + NP - h) % NP if d == 0 else (p + h) % NP
            qs = t ^ (t >> 1)
            sl = pl.ds(qs * qm + d * (qm // 2) + s * agm, agm)
            return pltpu.make_async_remote_copy(
                src_ref=out_ref.at[sl, :], dst_ref=out_ref.at[sl, :],
                send_sem=ag_ssem.at[d, h, s], recv_sem=ag_rsem.at[d, h, s],
                device_id=(rx, ry, my_z) if d == 0 else (lx, ly, my_z),
                device_id_type=pl.DeviceIdType.MESH)

        def ag_recv_desc(d, h, s):
            t = (p + NP - h - 1) % NP if d == 0 else (p + h + 1) % NP
            qs = t ^ (t >> 1)
            sl = pl.ds(qs * qm + d * (qm // 2) + s * agm, agm)
            return pltpu.make_async_remote_copy(
                src_ref=out_ref.at[sl, :], dst_ref=out_ref.at[sl, :],
                send_sem=ag_ssem.at[d, h, s], recv_sem=ag_rsem.at[d, h, s],
                device_id=(rx, ry, my_z) if d == 0 else (lx, ly, my_z),
                device_id_type=pl.DeviceIdType.MESH)

        sends = []
        for h in range(NP - 1):
            for s in range(AGK):
                for d in range(2):
                    if h > 0:
                        ag_recv_desc(d, h - 1, s).wait_recv()
                    dsc = ag_desc(d, h, s)
                    dsc.start()
                    sends.append(dsc)
        for s in range(AGK):
            for d in range(2):
                ag_recv_desc(d, NP - 2, s).wait_recv()

        for dsc in sends:
            dsc.wait_send()
        for j in range(K):
            @pl.when(my_z < NZ - 1)
            def _(j=j):
                pf_d[j].wait_send()

            @pl.when(my_z > 0)
            def _(j=j):
                sf_d[j].wait_send()

    return pl.pallas_call(
        body,
        out_shape=jax.ShapeDtypeStruct((m, n), jnp.bfloat16),
        in_specs=[pl.BlockSpec(memory_space=pltpu.VMEM)],
        out_specs=pl.BlockSpec(memory_space=pltpu.VMEM),
        scratch_shapes=[
            pltpu.VMEM((K, sm, n), jnp.bfloat16),
            pltpu.VMEM((K, sm, n), jnp.bfloat16),
            pltpu.SemaphoreType.DMA((K,)),
            pltpu.SemaphoreType.DMA((K,)),
            pltpu.SemaphoreType.DMA((K,)),
            pltpu.SemaphoreType.DMA((K,)),
            pltpu.SemaphoreType.DMA((2, NP - 1, AGK)),
            pltpu.SemaphoreType.DMA((2, NP - 1, AGK)),
        ],
        compiler_params=pltpu.CompilerParams(
            collective_id=0,
            vmem_limit_bytes=100 * 1024 * 1024,
        ),
    )(x)


# device time: 224452 ns/iter; 1.4665x vs baseline; 1.1209x over previous
import jax
import jax.numpy as jnp
from jax import lax
from jax.experimental import pallas as pl
from jax.experimental.pallas import tpu as pltpu

NZ = 4
NP = 4
K = 8
KH = K // 2


def kernel(x):
    x = x.astype(jnp.bfloat16)
    m, n = x.shape
    qm = m // NP
    sm = qm // K
    hm = sm // 2

    def body(x_ref, out_ref, pf, sf,
             pf_ssem, pf_rsem, sf_ssem, sf_rsem, ag_ssem, ag_rsem):
        my_x = lax.axis_index("x")
        my_y = lax.axis_index("y")
        my_z = lax.axis_index("z")
        q = 2 * my_x + my_y
        qbase = q * qm

        p = jnp.where(my_x == 0, my_y, 3 - my_y)

        def ring_coords(t):
            return (jnp.where(t >= 2, 1, 0),
                    jnp.where((t == 1) | (t == 2), 1, 0))

        rx, ry = ring_coords((p + 1) % NP)
        lx, ly = ring_coords((p + 3) % NP)
        zr = (my_z + 1) % NZ
        zl = (my_z + NZ - 1) % NZ

        barrier_sem = pltpu.get_barrier_semaphore()
        for dev in ((my_x, my_y, zr), (my_x, my_y, zl),
                    (rx, ry, my_z), (lx, ly, my_z)):
            pl.semaphore_signal(
                barrier_sem, inc=1,
                device_id=dev, device_id_type=pl.DeviceIdType.MESH,
            )
        pl.semaphore_wait(barrier_sem, 4)

        def rows(j):
            return pl.ds(qbase + j * sm, sm)

        is_head = my_z == 0
        is_tail = my_z == NZ - 1
        interior = jnp.logical_and(my_z > 0, my_z < NZ - 1)

        pf_d = [pltpu.make_async_remote_copy(
            src_ref=pf.at[j], dst_ref=pf.at[j],
            send_sem=pf_ssem.at[j], recv_sem=pf_rsem.at[j],
            device_id=(my_x, my_y, zr),
            device_id_type=pl.DeviceIdType.MESH) for j in range(K)]
        sf_d = [pltpu.make_async_remote_copy(
            src_ref=sf.at[j], dst_ref=sf.at[j],
            send_sem=sf_ssem.at[j], recv_sem=sf_rsem.at[j],
            device_id=(my_x, my_y, zl),
            device_id_type=pl.DeviceIdType.MESH) for j in range(K)]

        def ag_desc(d, h, j, inbound):
            off = h + 1 if inbound else h
            t = (p + NP - off) % NP if d == 0 else (p + off) % NP
            qs = t ^ (t >> 1)
            sl = pl.ds(qs * qm + j * sm + d * hm, hm)
            return pltpu.make_async_remote_copy(
                src_ref=out_ref.at[sl, :], dst_ref=out_ref.at[sl, :],
                send_sem=ag_ssem.at[d, h, j], recv_sem=ag_rsem.at[d, h, j],
                device_id=(rx, ry, my_z) if d == 0 else (lx, ly, my_z),
                device_id_type=pl.DeviceIdType.MESH)

        @pl.when(is_head)
        def _():
            for j in range(K):
                pf[j] = x_ref[rows(j), :]
                pf_d[j].start()

        @pl.when(is_tail)
        def _():
            for j in range(K):
                sf[j] = x_ref[rows(j), :]
                sf_d[j].start()

        for i in range(KH):
            ja, jb = i, KH + i

            @pl.when(my_z > 0)
            def _(j=ja):
                pf_d[j].wait_recv()
                pf[j] = pf[j] + x_ref[rows(j), :]

            @pl.when(interior)
            def _(j=ja):
                pf_d[j].start()

            @pl.when(my_z < NZ - 1)
            def _(j=jb):
                sf_d[j].wait_recv()
                sf[j] = sf[j] + x_ref[rows(j), :]

            @pl.when(interior)
            def _(j=jb):
                sf_d[j].start()

        def xy_step(i):
            for lag in range(NP):
                ii = i - lag
                if not 0 <= ii < KH:
                    continue
                for j in (ii, KH + ii):
                    for d in range(2):
                        if lag > 0:
                            ag_desc(d, lag - 1, j, inbound=True).wait_recv()
                        if lag < NP - 1:
                            ag_desc(d, lag, j, inbound=False).start()

        for i in range(KH + NP - 1):
            if i < KH:
                ja, jb = i, KH + i

                @pl.when(my_z < NZ - 1)
                def _(j=ja):
                    sf_d[j].wait_recv()
                    out_ref[rows(j), :] = pf[j] + sf[j]

                @pl.when(is_tail)
                def _(j=ja):
                    out_ref[rows(j), :] = pf[j]

                @pl.when(interior)
                def _(j=ja):
                    sf[j] = sf[j] + x_ref[rows(j), :]
                    sf_d[j].start()

                @pl.when(my_z > 0)
                def _(j=jb):
                    pf_d[j].wait_recv()
                    out_ref[rows(j), :] = pf[j] + sf[j]

                @pl.when(is_head)
                def _(j=jb):
                    out_ref[rows(j), :] = sf[j]

                @pl.when(interior)
                def _(j=jb):
                    pf[j] = pf[j] + x_ref[rows(j), :]
                    pf_d[j].start()

            xy_step(i)

        for h in range(NP - 1):
            for j in range(K):
                for d in range(2):
                    ag_desc(d, h, j, inbound=False).wait_send()
        for j in range(K):
            @pl.when(my_z < NZ - 1)
            def _(j=j):
                pf_d[j].wait_send()

            @pl.when(my_z > 0)
            def _(j=j):
                sf_d[j].wait_send()

    return pl.pallas_call(
        body,
        out_shape=jax.ShapeDtypeStruct((m, n), jnp.bfloat16),
        in_specs=[pl.BlockSpec(memory_space=pltpu.VMEM)],
        out_specs=pl.BlockSpec(memory_space=pltpu.VMEM),
        scratch_shapes=[
            pltpu.VMEM((K, sm, n), jnp.bfloat16),
            pltpu.VMEM((K, sm, n), jnp.bfloat16),
            pltpu.SemaphoreType.DMA((K,)),
            pltpu.SemaphoreType.DMA((K,)),
            pltpu.SemaphoreType.DMA((K,)),
            pltpu.SemaphoreType.DMA((K,)),
            pltpu.SemaphoreType.DMA((2, NP - 1, K)),
            pltpu.SemaphoreType.DMA((2, NP - 1, K)),
        ],
        compiler_params=pltpu.CompilerParams(
            collective_id=0,
            vmem_limit_bytes=100 * 1024 * 1024,
        ),
    )(x)


# device time: 219460 ns/iter; 1.4998x vs baseline; 1.0227x over previous
import jax
import jax.numpy as jnp
from jax import lax
from jax.experimental import pallas as pl
from jax.experimental.pallas import tpu as pltpu

NZ = 4
NP = 4
K = 4
KH = K // 2


def kernel(x):
    m, n = x.shape
    qm = m // NP
    sm = qm // K
    hm = sm // 2

    def body(x_ref, out_ref, pf, sf,
             pf_ssem, pf_rsem, sf_ssem, sf_rsem, ag_ssem, ag_rsem):
        my_x = lax.axis_index("x")
        my_y = lax.axis_index("y")
        my_z = lax.axis_index("z")
        q = 2 * my_x + my_y
        qbase = q * qm

        p = jnp.where(my_x == 0, my_y, 3 - my_y)

        def ring_coords(t):
            return (jnp.where(t >= 2, 1, 0),
                    jnp.where((t == 1) | (t == 2), 1, 0))

        rx, ry = ring_coords((p + 1) % NP)
        lx, ly = ring_coords((p + 3) % NP)
        zr = (my_z + 1) % NZ
        zl = (my_z + NZ - 1) % NZ

        barrier_sem = pltpu.get_barrier_semaphore()
        for dev in ((my_x, my_y, zr), (my_x, my_y, zl),
                    (rx, ry, my_z), (lx, ly, my_z)):
            pl.semaphore_signal(
                barrier_sem, inc=1,
                device_id=dev, device_id_type=pl.DeviceIdType.MESH,
            )
        pl.semaphore_wait(barrier_sem, 4)

        def rows(j):
            return pl.ds(qbase + j * sm, sm)

        def xv(j):
            return x_ref[rows(j), :].astype(jnp.bfloat16)

        is_head = my_z == 0
        is_tail = my_z == NZ - 1
        interior = jnp.logical_and(my_z > 0, my_z < NZ - 1)

        pf_d = [pltpu.make_async_remote_copy(
            src_ref=pf.at[j], dst_ref=pf.at[j],
            send_sem=pf_ssem.at[j], recv_sem=pf_rsem.at[j],
            device_id=(my_x, my_y, zr),
            device_id_type=pl.DeviceIdType.MESH) for j in range(K)]
        sf_d = [pltpu.make_async_remote_copy(
            src_ref=sf.at[j], dst_ref=sf.at[j],
            send_sem=sf_ssem.at[j], recv_sem=sf_rsem.at[j],
            device_id=(my_x, my_y, zl),
            device_id_type=pl.DeviceIdType.MESH) for j in range(K)]

        def ag_desc(d, h, j, inbound):
            off = h + 1 if inbound else h
            t = (p + NP - off) % NP if d == 0 else (p + off) % NP
            qs = t ^ (t >> 1)
            sl = pl.ds(qs * qm + j * sm + d * hm, hm)
            return pltpu.make_async_remote_copy(
                src_ref=out_ref.at[sl, :], dst_ref=out_ref.at[sl, :],
                send_sem=ag_ssem.at[d, h, j], recv_sem=ag_rsem.at[d, h, j],
                device_id=(rx, ry, my_z) if d == 0 else (lx, ly, my_z),
                device_id_type=pl.DeviceIdType.MESH)

        @pl.when(is_head)
        def _():
            for j in range(K):
                pf[j] = xv(j)
                pf_d[j].start()

        @pl.when(is_tail)
        def _():
            for j in range(K):
                sf[j] = xv(j)
                sf_d[j].start()

        for i in range(KH):
            ja, jb = i, KH + i

            @pl.when(my_z > 0)
            def _(j=ja):
                pf_d[j].wait_recv()
                pf[j] = pf[j] + xv(j)

            @pl.when(interior)
            def _(j=ja):
                pf_d[j].start()

            @pl.when(my_z < NZ - 1)
            def _(j=jb):
                sf_d[j].wait_recv()
                sf[j] = sf[j] + xv(j)

            @pl.when(interior)
            def _(j=jb):
                sf_d[j].start()

        def xy_step(i):
            for lag in range(NP):
                ii = i - lag
                if not 0 <= ii < KH:
                    continue
                for j in (ii, KH + ii):
                    for d in range(2):
                        if lag > 0:
                            ag_desc(d, lag - 1, j, inbound=True).wait_recv()
                        if lag < NP - 1:
                            ag_desc(d, lag, j, inbound=False).start()

        for i in range(KH + NP - 1):
            if i < KH:
                ja, jb = i, KH + i

                @pl.when(my_z < NZ - 1)
                def _(j=ja):
                    sf_d[j].wait_recv()
                    out_ref[rows(j), :] = pf[j] + sf[j]

                @pl.when(is_tail)
                def _(j=ja):
                    out_ref[rows(j), :] = pf[j]

                @pl.when(interior)
                def _(j=ja):
                    sf[j] = sf[j] + xv(j)
                    sf_d[j].start()

                @pl.when(my_z > 0)
                def _(j=jb):
                    pf_d[j].wait_recv()
                    out_ref[rows(j), :] = pf[j] + sf[j]

                @pl.when(is_head)
                def _(j=jb):
                    out_ref[rows(j), :] = sf[j]

                @pl.when(interior)
                def _(j=jb):
                    pf[j] = pf[j] + xv(j)
                    pf_d[j].start()

            xy_step(i)

        for h in range(NP - 1):
            for j in range(K):
                for d in range(2):
                    ag_desc(d, h, j, inbound=False).wait_send()
        for j in range(K):
            @pl.when(my_z < NZ - 1)
            def _(j=j):
                pf_d[j].wait_send()

            @pl.when(my_z > 0)
            def _(j=j):
                sf_d[j].wait_send()

    return pl.pallas_call(
        body,
        out_shape=jax.ShapeDtypeStruct((m, n), jnp.bfloat16),
        in_specs=[pl.BlockSpec(memory_space=pltpu.VMEM)],
        out_specs=pl.BlockSpec(memory_space=pltpu.VMEM),
        scratch_shapes=[
            pltpu.VMEM((K, sm, n), jnp.bfloat16),
            pltpu.VMEM((K, sm, n), jnp.bfloat16),
            pltpu.SemaphoreType.DMA((K,)),
            pltpu.SemaphoreType.DMA((K,)),
            pltpu.SemaphoreType.DMA((K,)),
            pltpu.SemaphoreType.DMA((K,)),
            pltpu.SemaphoreType.DMA((2, NP - 1, K)),
            pltpu.SemaphoreType.DMA((2, NP - 1, K)),
        ],
        compiler_params=pltpu.CompilerParams(
            collective_id=0,
            vmem_limit_bytes=100 * 1024 * 1024,
        ),
    )(x)
